# baseline (speedup 1.0000x reference)
import jax
import jax.numpy as jnp
from jax import lax
from jax.experimental import pallas as pl
from jax.experimental.pallas import tpu as pltpu
from jax.experimental.pallas import tpu_sc as plsc

GRID = 90
ROWS = GRID * GRID
D = 256
L = 16
NC = 2
NS = 16
NW = NC * NS
CHUNK = 16
CHUNKS_PER_W = 16
LAST_BASE = ROWS - CHUNK
DK = D // L
BIG_I32 = 2**30


def _som_bmu_sc(x_hbm, w_hbm, dist_out, info_out,
                x_v, buf, tmp_d, tmp_i, red_d, red_i,
                shared_d, shared_i):
    cid = lax.axis_index("c")
    sid = lax.axis_index("s")
    wid = sid * NC + cid

    pltpu.sync_copy(x_hbm, x_v)
    xs = [x_v[pl.ds(16 * k, 16)] for k in range(DK)]

    iota = lax.iota(jnp.int32, L)
    inf = jnp.full((L,), jnp.inf, jnp.float32)
    bitrev = (((iota & 1) << 3) | ((iota & 2) << 1)
              | ((iota & 4) >> 1) | ((iota & 8) >> 3))

    t0 = wid * CHUNKS_PER_W

    def row_acc(r):
        acc = jnp.zeros((L,), jnp.float32)
        for k in range(DK):
            v = buf[pl.ds(r * D + 16 * k, 16)] - xs[k]
            acc = acc + v * v
        return acc

    def merge_rows(r0, n):
        if n == 1:
            return row_acc(r0)
        a = merge_rows(r0, n // 2)
        b = merge_rows(r0 + n // 2, n // 2)
        d = L // n
        asum = a + a[iota ^ d]
        bsum = b + b[iota ^ d]
        return jnp.where((iota & d) == 0, asum, bsum)

    def chunk_body(t, carry):
        best, bidx = carry
        base = jnp.minimum((t0 + t) * CHUNK, LAST_BASE)
        off = pl.multiple_of(base * D, 1024)
        pltpu.sync_copy(w_hbm.at[pl.ds(off, CHUNK * D)], buf)
        s = merge_rows(0, CHUNK)
        rows = base + bitrev
        better = s < best
        best = jnp.where(better, s, best)
        bidx = jnp.where(better, rows, bidx)
        return best, bidx

    best, bidx = lax.fori_loop(0, CHUNKS_PER_W, chunk_body,
                               (inf, jnp.zeros((L,), jnp.int32)))

    for d in (8, 4, 2, 1):
        od = best[iota ^ d]
        oi = bidx[iota ^ d]
        take = (od < best) | ((od == best) & (oi < bidx))
        best = jnp.where(take, od, best)
        bidx = jnp.where(take, oi, bidx)

    tmp_d[:] = best
    tmp_i[:] = bidx
    soff = pl.multiple_of(sid * L, 16)
    pltpu.sync_copy(tmp_d, shared_d.at[pl.ds(soff, L)])
    pltpu.sync_copy(tmp_i, shared_i.at[pl.ds(soff, L)])
    plsc.subcore_barrier()

    @pl.when(sid == 0)
    def _():
        pltpu.sync_copy(shared_d, red_d)
        pltpu.sync_copy(shared_i, red_i)
        bd = red_d[pl.ds(0, L)]
        bi = red_i[pl.ds(0, L)]
        for srow in range(1, NS):
            d_ = red_d[pl.ds(srow * L, L)]
            i_ = red_i[pl.ds(srow * L, L)]
            take = (d_ < bd) | ((d_ == bd) & (i_ < bi))
            bd = jnp.where(take, d_, bd)
            bi = jnp.where(take, i_, bi)
        tmp_d[:] = bd
        tmp_i[:] = bi
        coff = pl.multiple_of(cid * L, 16)
        pltpu.sync_copy(tmp_d, dist_out.at[pl.ds(coff, L)])
        pltpu.sync_copy(tmp_i, info_out.at[pl.ds(coff, L)])


@jax.jit
def kernel(x, weights):
    w1d = weights.reshape(ROWS * D)
    mesh = plsc.VectorSubcoreMesh(core_axis_name="c", subcore_axis_name="s")
    dist_out, info_out = pl.kernel(
        _som_bmu_sc,
        mesh=mesh,
        out_type=[
            jax.ShapeDtypeStruct((NC * L,), jnp.float32),
            jax.ShapeDtypeStruct((NC * L,), jnp.int32),
        ],
        scratch_types=[
            pltpu.VMEM((D,), jnp.float32),
            pltpu.VMEM((CHUNK * D,), jnp.float32),
            pltpu.VMEM((L,), jnp.float32),
            pltpu.VMEM((L,), jnp.int32),
            pltpu.VMEM((NS * L,), jnp.float32),
            pltpu.VMEM((NS * L,), jnp.int32),
            pltpu.VMEM_SHARED((NS * L,), jnp.float32),
            pltpu.VMEM_SHARED((NS * L,), jnp.int32),
        ],
    )(x, w1d)

    d0, d1 = dist_out[0], dist_out[L]
    f0, f1 = info_out[0], info_out[L]
    take0 = (d0 < d1) | ((d0 == d1) & (f0 <= f1))
    flat = jnp.where(take0, f0, f1)
    return jnp.stack([flat // GRID, flat % GRID])


# speedup vs baseline: 1.2004x; 1.2004x over previous
import jax
import jax.numpy as jnp
from jax import lax
from jax.experimental import pallas as pl
from jax.experimental.pallas import tpu as pltpu
from jax.experimental.pallas import tpu_sc as plsc

GRID = 90
ROWS = GRID * GRID
D = 256
L = 16
NC = 2
NS = 16
NW = NC * NS
CHUNK = 16
SEGS = 4
SEG_ROWS = 64
SEG_ELEMS = SEG_ROWS * D
SEG_CHUNKS = SEG_ROWS // CHUNK
LAST_SEG = ROWS - SEG_ROWS
DK = D // L
BIG_I32 = 2**30


def _som_bmu_sc(x_hbm, w_hbm, dist_out, info_out,
                x_v, buf, tmp_d, tmp_i, red_d, red_i,
                shared_d, shared_i, sem0, sem1, sem2, sem3):
    cid = lax.axis_index("c")
    sid = lax.axis_index("s")
    wid = sid * NC + cid

    sems = [sem0, sem1, sem2, sem3]
    g0 = wid * SEGS

    def seg_base(s):
        return jnp.minimum((g0 + s) * SEG_ROWS, LAST_SEG)

    copies = []
    for s in range(SEGS):
        off = pl.multiple_of(seg_base(s) * D, 1024)
        copies.append(pltpu.async_copy(
            w_hbm.at[pl.ds(off, SEG_ELEMS)],
            buf.at[pl.ds(s * SEG_ELEMS, SEG_ELEMS)], sems[s]))

    pltpu.sync_copy(x_hbm, x_v)
    xs = [x_v[pl.ds(16 * k, 16)] for k in range(DK)]

    iota = lax.iota(jnp.int32, L)
    inf = jnp.full((L,), jnp.inf, jnp.float32)
    bitrev = (((iota & 1) << 3) | ((iota & 2) << 1)
              | ((iota & 4) >> 1) | ((iota & 8) >> 3))

    def row_acc(boff, r):
        acc = jnp.zeros((L,), jnp.float32)
        for k in range(DK):
            v = buf[pl.ds(boff + r * D + 16 * k, 16)] - xs[k]
            acc = acc + v * v
        return acc

    def merge_rows(boff, r0, n):
        if n == 1:
            return row_acc(boff, r0)
        a = merge_rows(boff, r0, n // 2)
        b = merge_rows(boff, r0 + n // 2, n // 2)
        d = L // n
        asum = a + a[iota ^ d]
        bsum = b + b[iota ^ d]
        return jnp.where((iota & d) == 0, asum, bsum)

    best = inf
    bidx = jnp.zeros((L,), jnp.int32)
    for s in range(SEGS):
        copies[s].wait()
        sbase = seg_base(s)

        def chunk_body(c, carry, s=s, sbase=sbase):
            best, bidx = carry
            boff = c * (CHUNK * D) + s * SEG_ELEMS
            sv = merge_rows(boff, 0, CHUNK)
            rows = sbase + c * CHUNK + bitrev
            better = sv < best
            best = jnp.where(better, sv, best)
            bidx = jnp.where(better, rows, bidx)
            return best, bidx

        best, bidx = lax.fori_loop(0, SEG_CHUNKS, chunk_body, (best, bidx))

    for d in (8, 4, 2, 1):
        od = best[iota ^ d]
        oi = bidx[iota ^ d]
        take = (od < best) | ((od == best) & (oi < bidx))
        best = jnp.where(take, od, best)
        bidx = jnp.where(take, oi, bidx)

    tmp_d[:] = best
    tmp_i[:] = bidx
    soff = pl.multiple_of(sid * L, 16)
    pltpu.sync_copy(tmp_d, shared_d.at[pl.ds(soff, L)])
    pltpu.sync_copy(tmp_i, shared_i.at[pl.ds(soff, L)])
    plsc.subcore_barrier()

    @pl.when(sid == 0)
    def _():
        pltpu.sync_copy(shared_d, red_d)
        pltpu.sync_copy(shared_i, red_i)
        bd = red_d[pl.ds(0, L)]
        bi = red_i[pl.ds(0, L)]
        for srow in range(1, NS):
            d_ = red_d[pl.ds(srow * L, L)]
            i_ = red_i[pl.ds(srow * L, L)]
            take = (d_ < bd) | ((d_ == bd) & (i_ < bi))
            bd = jnp.where(take, d_, bd)
            bi = jnp.where(take, i_, bi)
        tmp_d[:] = bd
        tmp_i[:] = bi
        coff = pl.multiple_of(cid * L, 16)
        pltpu.sync_copy(tmp_d, dist_out.at[pl.ds(coff, L)])
        pltpu.sync_copy(tmp_i, info_out.at[pl.ds(coff, L)])


@jax.jit
def kernel(x, weights):
    w1d = weights.reshape(ROWS * D)
    mesh = plsc.VectorSubcoreMesh(core_axis_name="c", subcore_axis_name="s")
    dist_out, info_out = pl.kernel(
        _som_bmu_sc,
        mesh=mesh,
        out_type=[
            jax.ShapeDtypeStruct((NC * L,), jnp.float32),
            jax.ShapeDtypeStruct((NC * L,), jnp.int32),
        ],
        scratch_types=[
            pltpu.VMEM((D,), jnp.float32),
            pltpu.VMEM((SEGS * SEG_ELEMS,), jnp.float32),
            pltpu.VMEM((L,), jnp.float32),
            pltpu.VMEM((L,), jnp.int32),
            pltpu.VMEM((NS * L,), jnp.float32),
            pltpu.VMEM((NS * L,), jnp.int32),
            pltpu.VMEM_SHARED((NS * L,), jnp.float32),
            pltpu.VMEM_SHARED((NS * L,), jnp.int32),
            pltpu.SemaphoreType.DMA,
            pltpu.SemaphoreType.DMA,
            pltpu.SemaphoreType.DMA,
            pltpu.SemaphoreType.DMA,
        ],
    )(x, w1d)

    d0, d1 = dist_out[0], dist_out[L]
    f0, f1 = info_out[0], info_out[L]
    take0 = (d0 < d1) | ((d0 == d1) & (f0 <= f1))
    flat = jnp.where(take0, f0, f1)
    return jnp.stack([flat // GRID, flat % GRID])


# speedup vs baseline: 1.3743x; 1.1449x over previous
import jax
import jax.numpy as jnp
from jax import lax
from jax.experimental import pallas as pl
from jax.experimental.pallas import tpu as pltpu
from jax.experimental.pallas import tpu_sc as plsc

GRID = 90
ROWS = GRID * GRID
D = 256
L = 16
NC = 2
NS = 16
NW = NC * NS
CHUNK = 16
PLANES_PER_W = 3
LAST_P0 = GRID - PLANES_PER_W
PLANE_CHUNKS = 6
LAST_JB = GRID - CHUNK
DK = D // L
BIG_I32 = 2**30


def _som_bmu_sc(x_hbm, w_hbm, dist_out, info_out,
                x_v, buf3, tmp_d, tmp_i, red_d, red_i,
                shared_d, shared_i, sem0, sem1, sem2):
    cid = lax.axis_index("c")
    sid = lax.axis_index("s")
    wid = sid * NC + cid

    sems = [sem0, sem1, sem2]
    p0 = jnp.minimum(wid * PLANES_PER_W, LAST_P0)

    copies = []
    for p in range(PLANES_PER_W):
        copies.append(pltpu.async_copy(w_hbm.at[p0 + p], buf3.at[p], sems[p]))

    pltpu.sync_copy(x_hbm, x_v)
    xs = [x_v[pl.ds(16 * k, 16)] for k in range(DK)]

    iota = lax.iota(jnp.int32, L)
    inf = jnp.full((L,), jnp.inf, jnp.float32)
    bitrev = (((iota & 1) << 3) | ((iota & 2) << 1)
              | ((iota & 4) >> 1) | ((iota & 8) >> 3))

    def row_acc(p, jb, r):
        acc = jnp.zeros((L,), jnp.float32)
        for k in range(DK):
            v = buf3[p, jb + r, pl.ds(16 * k, 16)] - xs[k]
            acc = acc + v * v
        return acc

    def merge_rows(p, jb, r0, n):
        if n == 1:
            return row_acc(p, jb, r0)
        a = merge_rows(p, jb, r0, n // 2)
        b = merge_rows(p, jb, r0 + n // 2, n // 2)
        d = L // n
        asum = a + a[iota ^ d]
        bsum = b + b[iota ^ d]
        return jnp.where((iota & d) == 0, asum, bsum)

    best = inf
    bidx = jnp.zeros((L,), jnp.int32)
    for p in range(PLANES_PER_W):
        copies[p].wait()
        prow = (p0 + p) * GRID

        def chunk_body(c, carry, p=p, prow=prow):
            best, bidx = carry
            jb = jnp.minimum(c * CHUNK, LAST_JB)
            sv = merge_rows(p, jb, 0, CHUNK)
            rows = prow + jb + bitrev
            better = sv < best
            best = jnp.where(better, sv, best)
            bidx = jnp.where(better, rows, bidx)
            return best, bidx

        best, bidx = lax.fori_loop(0, PLANE_CHUNKS, chunk_body, (best, bidx))

    for d in (8, 4, 2, 1):
        od = best[iota ^ d]
        oi = bidx[iota ^ d]
        take = (od < best) | ((od == best) & (oi < bidx))
        best = jnp.where(take, od, best)
        bidx = jnp.where(take, oi, bidx)

    tmp_d[:] = best
    tmp_i[:] = bidx
    soff = pl.multiple_of(sid * L, 16)
    pltpu.sync_copy(tmp_d, shared_d.at[pl.ds(soff, L)])
    pltpu.sync_copy(tmp_i, shared_i.at[pl.ds(soff, L)])
    plsc.subcore_barrier()

    @pl.when(sid == 0)
    def _():
        pltpu.sync_copy(shared_d, red_d)
        pltpu.sync_copy(shared_i, red_i)
        bd = red_d[pl.ds(0, L)]
        bi = red_i[pl.ds(0, L)]
        for srow in range(1, NS):
            d_ = red_d[pl.ds(srow * L, L)]
            i_ = red_i[pl.ds(srow * L, L)]
            take = (d_ < bd) | ((d_ == bd) & (i_ < bi))
            bd = jnp.where(take, d_, bd)
            bi = jnp.where(take, i_, bi)
        tmp_d[:] = bd
        tmp_i[:] = bi
        coff = pl.multiple_of(cid * L, 16)
        pltpu.sync_copy(tmp_d, dist_out.at[pl.ds(coff, L)])
        pltpu.sync_copy(tmp_i, info_out.at[pl.ds(coff, L)])


@jax.jit
def kernel(x, weights):
    mesh = plsc.VectorSubcoreMesh(core_axis_name="c", subcore_axis_name="s")
    dist_out, info_out = pl.kernel(
        _som_bmu_sc,
        mesh=mesh,
        out_type=[
            jax.ShapeDtypeStruct((NC * L,), jnp.float32),
            jax.ShapeDtypeStruct((NC * L,), jnp.int32),
        ],
        scratch_types=[
            pltpu.VMEM((D,), jnp.float32),
            pltpu.VMEM((PLANES_PER_W, GRID, D), jnp.float32),
            pltpu.VMEM((L,), jnp.float32),
            pltpu.VMEM((L,), jnp.int32),
            pltpu.VMEM((NS * L,), jnp.float32),
            pltpu.VMEM((NS * L,), jnp.int32),
            pltpu.VMEM_SHARED((NS * L,), jnp.float32),
            pltpu.VMEM_SHARED((NS * L,), jnp.int32),
            pltpu.SemaphoreType.DMA,
            pltpu.SemaphoreType.DMA,
            pltpu.SemaphoreType.DMA,
        ],
    )(x, weights)

    d0, d1 = dist_out[0], dist_out[L]
    f0, f1 = info_out[0], info_out[L]
    take0 = (d0 < d1) | ((d0 == d1) & (f0 <= f1))
    flat = jnp.where(take0, f0, f1)
    return jnp.stack([flat // GRID, flat % GRID])
